# baseline (speedup 1.0000x reference)
import functools

import jax
import jax.numpy as jnp
from jax import lax
from jax.experimental import pallas as pl
from jax.experimental.pallas import tpu as pltpu
from jax.experimental.pallas import tpu_sc as plsc

_NC = 2
_NS = 16
_NW = _NC * _NS
_L = 16
_K = 512
_ACC_W = 144


def _make_seg_sum(n_edges, n_nodes, dim):
  assert dim == 128
  nrw = -(-n_nodes // (8 * _NW)) * 8
  npad = nrw * _NW
  acc_rows = nrw + _L
  n_chunks = n_edges // _K
  n_groups = _K // _L
  nsel_cap = _K + _L

  mesh = plsc.VectorSubcoreMesh(core_axis_name="c", subcore_axis_name="s")

  @functools.partial(
      pl.kernel, mesh=mesh,
      out_type=jax.ShapeDtypeStruct((npad, dim), jnp.float32),
      compiler_params=pltpu.CompilerParams(needs_layout_passes=False),
      scratch_types=[
          pltpu.VMEM((_K,), jnp.int32),
          pltpu.VMEM((_K,), jnp.int32),
          pltpu.VMEM((nsel_cap,), jnp.int32),
          pltpu.VMEM((nsel_cap,), jnp.int32),
          pltpu.VMEM((nsel_cap, dim), jnp.float32),
          pltpu.VMEM((acc_rows, dim), jnp.float32),
          pltpu.VMEM((acc_rows + _L,), jnp.float32),
          pltpu.SemaphoreType.DMA,
      ],
  )
  def seg_sum(src_hbm, dst_hbm, x_hbm, zf_hbm, zc_hbm, out_hbm,
              dstc, srcc, sel_src, sel_lrow, rows, acc, cnt, sem):
    cid = lax.axis_index("c")
    sid = lax.axis_index("s")
    wid = sid * _NC + cid
    lo = wid * nrw
    pltpu.sync_copy(zf_hbm, acc)
    pltpu.sync_copy(zc_hbm, cnt)
    iot = lax.iota(jnp.int32, _L)
    zero_v = lax.broadcast_in_dim(jnp.int32(0), (_L,), ())
    neg1 = lax.broadcast_in_dim(jnp.int32(-1), (_L,), ())
    nrw_v = lax.broadcast_in_dim(jnp.int32(nrw), (_L,), ())
    trash = nrw_v
    one_f = lax.broadcast_in_dim(jnp.float32(1.0), (_L,), ())
    one_v = lax.broadcast_in_dim(jnp.int32(1), (_L,), ())
    zero_f = lax.broadcast_in_dim(jnp.float32(0.0), (_L,), ())
    one_hot0 = lax.select(lax.eq(iot, zero_v), one_f, zero_f)
    lo_v = lax.broadcast_in_dim(lo, (_L,), ())

    def chunk_body(ci, carry):
      off = ci * _K
      pltpu.sync_copy(dst_hbm.at[pl.ds(off, _K)], dstc)
      pltpu.sync_copy(src_hbm.at[pl.ds(off, _K)], srcc)
      for g in range(n_groups + 1):
        sel_src[pl.ds(g * _L, _L)] = neg1
        sel_lrow[pl.ds(g * _L, _L)] = trash

      def scan_g(g, ptr):
        dvec = dstc[pl.ds(g * _L, _L)]
        svec = srcc[pl.ds(g * _L, _L)]
        lrow = dvec - lo_v
        mask = (lrow >= zero_v) & (lrow < nrw_v)
        mi = lax.select(mask, one_v, zero_v)
        incl = plsc.cumsum(mi)
        ptr_v = lax.broadcast_in_dim(ptr, (_L,), ())
        pos = ptr_v + incl - mi
        plsc.store_scatter(sel_src, [pos], svec, mask=mask)
        plsc.store_scatter(sel_lrow, [pos], lrow, mask=mask)
        return ptr + incl[_L - 1]

      n_sel = lax.fori_loop(0, n_groups, scan_g, 0)

      pltpu.async_copy(
          x_hbm.at[plsc.Indices(sel_src, ignored_value=-1)], rows, sem
      ).wait()

      def acc_g(g, carry2):
        lvec = sel_lrow[pl.ds(g * _L, _L)]
        for lane in range(_L):
          r = lvec[lane]
          e = g * _L + lane
          for f in range(8):
            c = f * _L
            acc[r, pl.ds(c, _L)] = acc[r, pl.ds(c, _L)] + rows[e, pl.ds(c, _L)]
          cnt[pl.ds(r, _L)] = cnt[pl.ds(r, _L)] + one_hot0
        return carry2

      lax.fori_loop(0, (n_sel + _L - 1) // _L, acc_g, 0)
      return carry

    lax.fori_loop(0, n_chunks, chunk_body, 0)

    def mean_r(r, carry):
      c0 = cnt[pl.ds(r, _L)][0]
      rv = lax.div(one_f, jnp.maximum(lax.broadcast_in_dim(c0, (_L,), ()),
                                      one_f))
      for f in range(8):
        c = f * _L
        acc[r, pl.ds(c, _L)] = acc[r, pl.ds(c, _L)] * rv
      return carry

    lax.fori_loop(0, nrw, mean_r, 0)
    pltpu.sync_copy(acc.at[pl.ds(0, nrw)], out_hbm.at[pl.ds(lo, nrw)])

  return seg_sum, npad


_R = 1000


def _dense1_body(s_ref, x_ref, w1l_ref, b1_ref, w1r_ref,
                 w2l_ref, b2_ref, w2r_ref, p_ref, q_ref):
  mean = s_ref[...]
  h = jnp.dot(mean, w1l_ref[...], preferred_element_type=jnp.float32)
  h = h + b1_ref[...]
  h = h + jnp.dot(x_ref[...], w1r_ref[...], preferred_element_type=jnp.float32)
  h = jnp.maximum(h, 0.0)
  p_ref[...] = jnp.dot(h, w2l_ref[...], preferred_element_type=jnp.float32)
  q_ref[...] = (jnp.dot(h, w2r_ref[...], preferred_element_type=jnp.float32)
                + b2_ref[...])


def _out_body(t_ref, q_ref, o_ref):
  z = t_ref[...] + q_ref[...]
  m = jnp.max(z, axis=1, keepdims=True)
  lse = jnp.log(jnp.sum(jnp.exp(z - m), axis=1, keepdims=True))
  o_ref[...] = z - m - lse


def kernel(x, edge_index, W1l, b1, W1r, W2l, b2, W2r):
  n_nodes, dim_in = x.shape
  dim_h = W1l.shape[0]
  dim_out = W2l.shape[0]
  n_edges = edge_index.shape[1]

  src = edge_index[0].astype(jnp.int32)
  dst = edge_index[1].astype(jnp.int32)

  seg, npad = _make_seg_sum(n_edges, n_nodes, dim_in)
  nrw = npad // _NW
  zeros_acc = jnp.zeros((nrw + _L, dim_in), jnp.float32)
  zeros_cnt = jnp.zeros((nrw + 2 * _L,), jnp.float32)
  S = seg(src, dst, x, zeros_acc, zeros_cnt)

  grid = n_nodes // _R
  p, q = pl.pallas_call(
      _dense1_body,
      grid=(grid,),
      in_specs=[
          pl.BlockSpec((_R, dim_in), lambda i: (i, 0)),
          pl.BlockSpec((_R, dim_in), lambda i: (i, 0)),
          pl.BlockSpec((dim_in, dim_h), lambda i: (0, 0)),
          pl.BlockSpec((1, dim_h), lambda i: (0, 0)),
          pl.BlockSpec((dim_in, dim_h), lambda i: (0, 0)),
          pl.BlockSpec((dim_h, dim_out), lambda i: (0, 0)),
          pl.BlockSpec((1, dim_out), lambda i: (0, 0)),
          pl.BlockSpec((dim_h, dim_out), lambda i: (0, 0)),
      ],
      out_specs=[
          pl.BlockSpec((_R, dim_out), lambda i: (i, 0)),
          pl.BlockSpec((_R, dim_out), lambda i: (i, 0)),
      ],
      out_shape=[
          jax.ShapeDtypeStruct((n_nodes, dim_out), jnp.float32),
          jax.ShapeDtypeStruct((n_nodes, dim_out), jnp.float32),
      ],
  )(S, x, W1l.T, b1.reshape(1, -1), W1r.T, W2l.T, b2.reshape(1, -1), W2r.T)

  T = seg(src, dst, p, zeros_acc, zeros_cnt)

  out = pl.pallas_call(
      _out_body,
      grid=(grid,),
      in_specs=[
          pl.BlockSpec((_R, dim_out), lambda i: (i, 0)),
          pl.BlockSpec((_R, dim_out), lambda i: (i, 0)),
      ],
      out_specs=pl.BlockSpec((_R, dim_out), lambda i: (i, 0)),
      out_shape=jax.ShapeDtypeStruct((n_nodes, dim_out), jnp.float32),
  )(T, q)
  return out


# speedup vs baseline: 2.9598x; 2.9598x over previous
import functools

import jax
import jax.numpy as jnp
from jax import lax
from jax.experimental import pallas as pl
from jax.experimental.pallas import tpu as pltpu
from jax.experimental.pallas import tpu_sc as plsc

_NC = 2
_NS = 16
_NW = _NC * _NS
_L = 16
_K = 1280
_GB = 128


def _make_seg_sum(n_edges, n_nodes, dim):
  assert dim == 128
  nrw = -(-n_nodes // (8 * _NW)) * 8
  npad = nrw * _NW
  acc_rows = nrw + _L
  n_chunks = n_edges // _K
  n_groups = _K // _L
  nsel_cap = _K + _GB + _L

  mesh = plsc.VectorSubcoreMesh(core_axis_name="c", subcore_axis_name="s")

  @functools.partial(
      pl.kernel, mesh=mesh,
      out_type=jax.ShapeDtypeStruct((npad, dim), jnp.float32),
      compiler_params=pltpu.CompilerParams(needs_layout_passes=False),
      scratch_types=[
          pltpu.VMEM((2, 2, _K), jnp.int32),
          pltpu.VMEM((nsel_cap,), jnp.int32),
          pltpu.VMEM((nsel_cap,), jnp.int32),
          pltpu.VMEM((_GB, dim), jnp.float32),
          pltpu.VMEM((acc_rows, dim), jnp.float32),
          pltpu.VMEM((acc_rows + _L,), jnp.float32),
          pltpu.SemaphoreType.DMA,
          pltpu.SemaphoreType.DMA,
      ],
  )
  def seg_sum(eidx_hbm, x_hbm, zf_hbm, zc_hbm, out_hbm,
              ebuf, sel_src, sel_lrow, rows, acc, cnt, gsem, psem):
    cid = lax.axis_index("c")
    sid = lax.axis_index("s")
    wid = sid * _NC + cid
    lo = wid * nrw
    pltpu.sync_copy(zf_hbm, acc)
    pltpu.sync_copy(zc_hbm, cnt)
    iot = lax.iota(jnp.int32, _L)
    zero_v = lax.broadcast_in_dim(jnp.int32(0), (_L,), ())
    one_v = lax.broadcast_in_dim(jnp.int32(1), (_L,), ())
    neg1 = lax.broadcast_in_dim(jnp.int32(-1), (_L,), ())
    nrw_v = lax.broadcast_in_dim(jnp.int32(nrw), (_L,), ())
    one_f = lax.broadcast_in_dim(jnp.float32(1.0), (_L,), ())
    zero_f = lax.broadcast_in_dim(jnp.float32(0.0), (_L,), ())
    one_hot0 = lax.select(lax.eq(iot, zero_v), one_f, zero_f)
    lo_v = lax.broadcast_in_dim(lo, (_L,), ())

    pltpu.sync_copy(eidx_hbm.at[:, pl.ds(0, _K)], ebuf.at[0])

    def chunk_body(ci, carry):
      p = lax.rem(ci, 2)

      @pl.when(ci + 1 < n_chunks)
      def _():
        pltpu.async_copy(eidx_hbm.at[:, pl.ds((ci + 1) * _K, _K)],
                         ebuf.at[lax.rem(ci + 1, 2)], psem)

      def scan_g(g, ptr):
        svec = ebuf[p, 0, pl.ds(g * _L, _L)]
        dvec = ebuf[p, 1, pl.ds(g * _L, _L)]
        lrow = dvec - lo_v
        mask = (lrow >= zero_v) & (lrow < nrw_v)
        mi = lax.select(mask, one_v, zero_v)
        incl = plsc.cumsum(mi)
        ptr_v = lax.broadcast_in_dim(ptr, (_L,), ())
        pos = ptr_v + incl - mi
        plsc.store_scatter(sel_src, [pos], svec, mask=mask)
        plsc.store_scatter(sel_lrow, [pos], lrow, mask=mask)
        return ptr + incl[_L - 1]

      n_sel = lax.fori_loop(0, n_groups, scan_g, 0)

      for t in range(_GB // _L):
        sel_src[pl.ds(n_sel + t * _L, _L)] = neg1
        sel_lrow[pl.ds(n_sel + t * _L, _L)] = nrw_v

      def sb_body(b, carry2):
        pltpu.async_copy(
            x_hbm.at[plsc.Indices(sel_src.at[pl.ds(b * _GB, _GB)],
                                  ignored_value=-1)],
            rows, gsem,
        ).wait()
        rem = n_sel - b * _GB
        ngr = jnp.minimum((rem + _L - 1) // _L, _GB // _L)

        def acc_g(g, carry3):
          lvec = sel_lrow[pl.ds(b * _GB + g * _L, _L)]
          for lane in range(_L):
            r = lvec[lane]
            e = g * _L + lane
            for f in range(8):
              c = f * _L
              acc[r, pl.ds(c, _L)] = (acc[r, pl.ds(c, _L)]
                                      + rows[e, pl.ds(c, _L)])
            cnt[pl.ds(r, _L)] = cnt[pl.ds(r, _L)] + one_hot0
          return carry3

        lax.fori_loop(0, ngr, acc_g, 0)
        return carry2

      lax.fori_loop(0, (n_sel + _GB - 1) // _GB, sb_body, 0)

      @pl.when(ci + 1 < n_chunks)
      def _():
        pltpu.make_async_copy(eidx_hbm.at[:, pl.ds((ci + 1) * _K, _K)],
                              ebuf.at[lax.rem(ci + 1, 2)], psem).wait()

      return carry

    lax.fori_loop(0, n_chunks, chunk_body, 0)

    def mean_r(r, carry):
      c0 = cnt[pl.ds(r, _L)][0]
      rv = lax.div(one_f, jnp.maximum(lax.broadcast_in_dim(c0, (_L,), ()),
                                      one_f))
      for f in range(8):
        c = f * _L
        acc[r, pl.ds(c, _L)] = acc[r, pl.ds(c, _L)] * rv
      return carry

    lax.fori_loop(0, nrw, mean_r, 0)
    pltpu.sync_copy(acc.at[pl.ds(0, nrw)], out_hbm.at[pl.ds(lo, nrw)])

  return seg_sum, npad


_R = 1000


def _dense1_body(s_ref, x_ref, w1l_ref, b1_ref, w1r_ref,
                 w2l_ref, b2_ref, w2r_ref, p_ref, q_ref):
  mean = s_ref[...]
  h = jnp.dot(mean, w1l_ref[...], preferred_element_type=jnp.float32)
  h = h + b1_ref[...]
  h = h + jnp.dot(x_ref[...], w1r_ref[...], preferred_element_type=jnp.float32)
  h = jnp.maximum(h, 0.0)
  p_ref[...] = jnp.dot(h, w2l_ref[...], preferred_element_type=jnp.float32)
  q_ref[...] = (jnp.dot(h, w2r_ref[...], preferred_element_type=jnp.float32)
                + b2_ref[...])


def _out_body(t_ref, q_ref, o_ref):
  z = t_ref[...] + q_ref[...]
  m = jnp.max(z, axis=1, keepdims=True)
  lse = jnp.log(jnp.sum(jnp.exp(z - m), axis=1, keepdims=True))
  o_ref[...] = z - m - lse


def kernel(x, edge_index, W1l, b1, W1r, W2l, b2, W2r):
  n_nodes, dim_in = x.shape
  dim_h = W1l.shape[0]
  dim_out = W2l.shape[0]
  n_edges = edge_index.shape[1]

  eidx = edge_index.astype(jnp.int32)

  seg, npad = _make_seg_sum(n_edges, n_nodes, dim_in)
  nrw = npad // _NW
  zeros_acc = jnp.zeros((nrw + _L, dim_in), jnp.float32)
  zeros_cnt = jnp.zeros((nrw + 2 * _L,), jnp.float32)
  S = seg(eidx, x, zeros_acc, zeros_cnt)

  grid = n_nodes // _R
  p, q = pl.pallas_call(
      _dense1_body,
      grid=(grid,),
      in_specs=[
          pl.BlockSpec((_R, dim_in), lambda i: (i, 0)),
          pl.BlockSpec((_R, dim_in), lambda i: (i, 0)),
          pl.BlockSpec((dim_in, dim_h), lambda i: (0, 0)),
          pl.BlockSpec((1, dim_h), lambda i: (0, 0)),
          pl.BlockSpec((dim_in, dim_h), lambda i: (0, 0)),
          pl.BlockSpec((dim_h, dim_out), lambda i: (0, 0)),
          pl.BlockSpec((1, dim_out), lambda i: (0, 0)),
          pl.BlockSpec((dim_h, dim_out), lambda i: (0, 0)),
      ],
      out_specs=[
          pl.BlockSpec((_R, dim_out), lambda i: (i, 0)),
          pl.BlockSpec((_R, dim_out), lambda i: (i, 0)),
      ],
      out_shape=[
          jax.ShapeDtypeStruct((n_nodes, dim_out), jnp.float32),
          jax.ShapeDtypeStruct((n_nodes, dim_out), jnp.float32),
      ],
  )(S, x, W1l.T, b1.reshape(1, -1), W1r.T, W2l.T, b2.reshape(1, -1), W2r.T)

  T = seg(eidx, p, zeros_acc, zeros_cnt)

  out = pl.pallas_call(
      _out_body,
      grid=(grid,),
      in_specs=[
          pl.BlockSpec((_R, dim_out), lambda i: (i, 0)),
          pl.BlockSpec((_R, dim_out), lambda i: (i, 0)),
      ],
      out_specs=pl.BlockSpec((_R, dim_out), lambda i: (i, 0)),
      out_shape=jax.ShapeDtypeStruct((n_nodes, dim_out), jnp.float32),
  )(T, q)
  return out


# speedup vs baseline: 4.2506x; 1.4361x over previous
import functools

import jax
import jax.numpy as jnp
from jax import lax
from jax.experimental import pallas as pl
from jax.experimental.pallas import tpu as pltpu
from jax.experimental.pallas import tpu_sc as plsc

_NC = 2
_NS = 16
_NW = _NC * _NS
_L = 16
_K = 2560
_GB = 128


def _make_seg_sum(n_edges, n_nodes, dim):
  assert dim == 128
  nrw = -(-n_nodes // (8 * _NW)) * 8
  npad = nrw * _NW
  acc_rows = nrw + _L
  n_chunks = n_edges // _K
  n_groups = _K // _L
  nsel_cap = _K + _GB + _L

  mesh = plsc.VectorSubcoreMesh(core_axis_name="c", subcore_axis_name="s")

  @functools.partial(
      pl.kernel, mesh=mesh,
      out_type=jax.ShapeDtypeStruct((npad, dim), jnp.float32),
      compiler_params=pltpu.CompilerParams(needs_layout_passes=False),
      scratch_types=[
          pltpu.VMEM((2, 2, _K), jnp.int32),
          pltpu.VMEM((nsel_cap,), jnp.int32),
          pltpu.VMEM((nsel_cap,), jnp.int32),
          pltpu.VMEM((_GB, dim), jnp.float32),
          pltpu.VMEM((acc_rows, dim), jnp.float32),
          pltpu.VMEM((acc_rows + _L,), jnp.float32),
          pltpu.SemaphoreType.DMA,
          pltpu.SemaphoreType.DMA,
      ],
  )
  def seg_sum(eidx_hbm, x_hbm, zf_hbm, zc_hbm, out_hbm,
              ebuf, sel_src, sel_lrow, rows, acc, cnt, gsem, psem):
    cid = lax.axis_index("c")
    sid = lax.axis_index("s")
    wid = sid * _NC + cid
    lo = wid * nrw
    pltpu.sync_copy(zf_hbm, acc)
    pltpu.sync_copy(zc_hbm, cnt)
    iot = lax.iota(jnp.int32, _L)
    zero_v = lax.broadcast_in_dim(jnp.int32(0), (_L,), ())
    one_v = lax.broadcast_in_dim(jnp.int32(1), (_L,), ())
    neg1 = lax.broadcast_in_dim(jnp.int32(-1), (_L,), ())
    nrw_v = lax.broadcast_in_dim(jnp.int32(nrw), (_L,), ())
    one_f = lax.broadcast_in_dim(jnp.float32(1.0), (_L,), ())
    zero_f = lax.broadcast_in_dim(jnp.float32(0.0), (_L,), ())
    one_hot0 = lax.select(lax.eq(iot, zero_v), one_f, zero_f)
    lo_v = lax.broadcast_in_dim(lo, (_L,), ())

    pltpu.sync_copy(eidx_hbm.at[:, pl.ds(0, _K)], ebuf.at[0])

    def chunk_body(ci, carry):
      p = lax.rem(ci, 2)

      @pl.when(ci + 1 < n_chunks)
      def _():
        pltpu.async_copy(eidx_hbm.at[:, pl.ds((ci + 1) * _K, _K)],
                         ebuf.at[lax.rem(ci + 1, 2)], psem)

      def scan_g(g, ptr):
        svec = ebuf[p, 0, pl.ds(g * _L, _L)]
        dvec = ebuf[p, 1, pl.ds(g * _L, _L)]
        lrow = dvec - lo_v
        mask = (lrow >= zero_v) & (lrow < nrw_v)
        mi = lax.select(mask, one_v, zero_v)
        incl = plsc.cumsum(mi)
        ptr_v = lax.broadcast_in_dim(ptr, (_L,), ())
        pos = ptr_v + incl - mi
        plsc.store_scatter(sel_src, [pos], svec, mask=mask)
        plsc.store_scatter(sel_lrow, [pos], lrow, mask=mask)
        return ptr + incl[_L - 1]

      n_sel = lax.fori_loop(0, n_groups, scan_g, 0)

      for t in range(_GB // _L):
        sel_src[pl.ds(n_sel + t * _L, _L)] = neg1
        sel_lrow[pl.ds(n_sel + t * _L, _L)] = nrw_v

      def sb_body(b, carry2):
        pltpu.async_copy(
            x_hbm.at[plsc.Indices(sel_src.at[pl.ds(b * _GB, _GB)],
                                  ignored_value=-1)],
            rows, gsem,
        ).wait()
        rem = n_sel - b * _GB
        ngr = jnp.minimum((rem + _L - 1) // _L, _GB // _L)

        def acc_g(g, carry3):
          lvec = sel_lrow[pl.ds(b * _GB + g * _L, _L)]
          for lane in range(_L):
            r = lvec[lane]
            e = g * _L + lane
            for f in range(8):
              c = f * _L
              plsc.addupdate(acc.at[r, pl.ds(c, _L)], rows[e, pl.ds(c, _L)])
            plsc.addupdate(cnt.at[pl.ds(r, _L)], one_hot0)
          return carry3

        lax.fori_loop(0, ngr, acc_g, 0)
        return carry2

      lax.fori_loop(0, (n_sel + _GB - 1) // _GB, sb_body, 0)

      @pl.when(ci + 1 < n_chunks)
      def _():
        pltpu.make_async_copy(eidx_hbm.at[:, pl.ds((ci + 1) * _K, _K)],
                              ebuf.at[lax.rem(ci + 1, 2)], psem).wait()

      return carry

    lax.fori_loop(0, n_chunks, chunk_body, 0)

    def mean_r(r, carry):
      c0 = cnt[pl.ds(r, _L)][0]
      rv = lax.div(one_f, jnp.maximum(lax.broadcast_in_dim(c0, (_L,), ()),
                                      one_f))
      for f in range(8):
        c = f * _L
        acc[r, pl.ds(c, _L)] = acc[r, pl.ds(c, _L)] * rv
      return carry

    lax.fori_loop(0, nrw, mean_r, 0)
    pltpu.sync_copy(acc.at[pl.ds(0, nrw)], out_hbm.at[pl.ds(lo, nrw)])

  return seg_sum, npad


_R = 1000


def _dense1_body(s_ref, x_ref, w1l_ref, b1_ref, w1r_ref,
                 w2l_ref, b2_ref, w2r_ref, p_ref, q_ref):
  mean = s_ref[...]
  h = jnp.dot(mean, w1l_ref[...], preferred_element_type=jnp.float32)
  h = h + b1_ref[...]
  h = h + jnp.dot(x_ref[...], w1r_ref[...], preferred_element_type=jnp.float32)
  h = jnp.maximum(h, 0.0)
  p_ref[...] = jnp.dot(h, w2l_ref[...], preferred_element_type=jnp.float32)
  q_ref[...] = (jnp.dot(h, w2r_ref[...], preferred_element_type=jnp.float32)
                + b2_ref[...])


def _out_body(t_ref, q_ref, o_ref):
  z = t_ref[...] + q_ref[...]
  m = jnp.max(z, axis=1, keepdims=True)
  lse = jnp.log(jnp.sum(jnp.exp(z - m), axis=1, keepdims=True))
  o_ref[...] = z - m - lse


def kernel(x, edge_index, W1l, b1, W1r, W2l, b2, W2r):
  n_nodes, dim_in = x.shape
  dim_h = W1l.shape[0]
  dim_out = W2l.shape[0]
  n_edges = edge_index.shape[1]

  eidx = edge_index.astype(jnp.int32)

  seg, npad = _make_seg_sum(n_edges, n_nodes, dim_in)
  nrw = npad // _NW
  zeros_acc = jnp.zeros((nrw + _L, dim_in), jnp.float32)
  zeros_cnt = jnp.zeros((nrw + 2 * _L,), jnp.float32)
  S = seg(eidx, x, zeros_acc, zeros_cnt)

  grid = n_nodes // _R
  p, q = pl.pallas_call(
      _dense1_body,
      grid=(grid,),
      in_specs=[
          pl.BlockSpec((_R, dim_in), lambda i: (i, 0)),
          pl.BlockSpec((_R, dim_in), lambda i: (i, 0)),
          pl.BlockSpec((dim_in, dim_h), lambda i: (0, 0)),
          pl.BlockSpec((1, dim_h), lambda i: (0, 0)),
          pl.BlockSpec((dim_in, dim_h), lambda i: (0, 0)),
          pl.BlockSpec((dim_h, dim_out), lambda i: (0, 0)),
          pl.BlockSpec((1, dim_out), lambda i: (0, 0)),
          pl.BlockSpec((dim_h, dim_out), lambda i: (0, 0)),
      ],
      out_specs=[
          pl.BlockSpec((_R, dim_out), lambda i: (i, 0)),
          pl.BlockSpec((_R, dim_out), lambda i: (i, 0)),
      ],
      out_shape=[
          jax.ShapeDtypeStruct((n_nodes, dim_out), jnp.float32),
          jax.ShapeDtypeStruct((n_nodes, dim_out), jnp.float32),
      ],
  )(S, x, W1l.T, b1.reshape(1, -1), W1r.T, W2l.T, b2.reshape(1, -1), W2r.T)

  T = seg(eidx, p, zeros_acc, zeros_cnt)

  out = pl.pallas_call(
      _out_body,
      grid=(grid,),
      in_specs=[
          pl.BlockSpec((_R, dim_out), lambda i: (i, 0)),
          pl.BlockSpec((_R, dim_out), lambda i: (i, 0)),
      ],
      out_specs=pl.BlockSpec((_R, dim_out), lambda i: (i, 0)),
      out_shape=jax.ShapeDtypeStruct((n_nodes, dim_out), jnp.float32),
  )(T, q)
  return out
